# baseline (speedup 1.0000x reference)
import jax
import jax.numpy as jnp
from jax.experimental import pallas as pl

B = 16
S = 2048
N_HASHES = 8
S_T = 2048


def _probe(bkt_ref):
    bkt_ref[...] = jnp.zeros_like(bkt_ref)


@jax.jit
def kernel(qk, v, random_rotations):
    bkt = pl.pallas_call(
        _probe,
        grid=(B, S // S_T),
        out_specs=pl.BlockSpec((1, N_HASHES, S_T), lambda b, s: (b, 0, s)),
        out_shape=jax.ShapeDtypeStruct((B, N_HASHES, S), jnp.int32),
    )()
    return bkt
